# Initial kernel scaffold; baseline (speedup 1.0000x reference)
import jax
import jax.numpy as jnp
from jax import lax
from jax.experimental import pallas as pl
from jax.experimental.pallas import tpu as pltpu
from jax.experimental.pallas import tpu_sc as plsc

N = 10000
E = 160000
D_IN = 256
HALF = 128
NC, NS = 2, 16
K_CH = 128
CH_PER_TILE = 80
E_TILE = CH_PER_TILE * K_CH
E_PAD = NS * E_TILE
NPAD = 10240
ROWS_PT = NPAD // NS
DUMMY = N

_MESH = plsc.VectorSubcoreMesh(
    core_axis_name="c", subcore_axis_name="s", num_cores=NC, num_subcores=NS
)


def _deg_body(col_hbm, out_hbm, cbuf, dloc):
    c = lax.axis_index("c")
    s = lax.axis_index("s")
    wid = s * NC + c
    n_my = E_TILE // NC
    pltpu.sync_copy(col_hbm.at[pl.ds(s * E_TILE + c * n_my, n_my)], cbuf)

    def zero(i, _):
        dloc[pl.ds(i * 16, 16)] = jnp.zeros((16,), jnp.float32)
        return 0

    lax.fori_loop(0, NPAD // 16, zero, 0)
    ones = jnp.ones((16,), jnp.float32)

    def scat(i, _):
        idx = cbuf[pl.ds(i * 16, 16)]
        plsc.addupdate_scatter(dloc, [idx], ones)
        return 0

    lax.fori_loop(0, n_my // 16, scat, 0)
    pltpu.sync_copy(dloc, out_hbm.at[wid])


_deg_call = pl.kernel(
    _deg_body,
    out_type=jax.ShapeDtypeStruct((NC * NS, NPAD), jnp.float32),
    mesh=_MESH,
    scratch_types=[
        pltpu.VMEM((E_TILE // NC,), jnp.int32),
        pltpu.VMEM((NPAD,), jnp.float32),
    ],
)


def _hop_body(g_hbm, row_hbm, col_hbm, out_hbm, rowbuf, colbuf, gbuf, dsem, acc):
    c = lax.axis_index("c")
    s = lax.axis_index("s")
    pltpu.sync_copy(
        g_hbm.at[pl.ds(c * NPAD + s * ROWS_PT, ROWS_PT)],
        acc.at[pl.ds(s * ROWS_PT, ROWS_PT)],
    )
    pltpu.sync_copy(row_hbm.at[pl.ds(s * E_TILE, E_TILE)], rowbuf)
    pltpu.sync_copy(col_hbm.at[s], colbuf)
    off = c * NPAD

    def adj(i, _):
        rowbuf[pl.ds(i * 16, 16)] = rowbuf[pl.ds(i * 16, 16)] + off
        return 0

    lax.fori_loop(0, E_TILE // 16, adj, 0)
    plsc.subcore_barrier()

    def step(j, _):
        pltpu.async_copy(
            g_hbm.at[rowbuf.at[pl.ds(j * K_CH, K_CH)]], gbuf, dsem
        ).wait()
        pltpu.sync_copy(gbuf, acc.at[colbuf.at[j]], add=True)
        return 0

    lax.fori_loop(0, CH_PER_TILE, step, 0)
    plsc.subcore_barrier()
    pltpu.sync_copy(
        acc.at[pl.ds(s * ROWS_PT, ROWS_PT)],
        out_hbm.at[pl.ds(c * NPAD + s * ROWS_PT, ROWS_PT)],
    )


_hop_call = pl.kernel(
    _hop_body,
    out_type=jax.ShapeDtypeStruct((NC * NPAD, HALF), jnp.float32),
    mesh=_MESH,
    scratch_types=[
        pltpu.VMEM((E_TILE,), jnp.int32),
        pltpu.VMEM((CH_PER_TILE, K_CH), jnp.int32),
        pltpu.VMEM((K_CH, HALF), jnp.float32),
        pltpu.SemaphoreType.DMA,
        pltpu.VMEM_SHARED((NPAD, HALF), jnp.float32),
    ],
)


def _lin_body(x_ref, w_ref, degp_ref, g_ref):
    h = lax.dot_general(
        x_ref[...], w_ref[...], (((1,), (1,)), ((), ())),
        preferred_element_type=jnp.float32,
    )
    deg = jnp.sum(degp_ref[...], axis=0) + 1.0
    dinv = lax.rsqrt(deg)
    g_ref[0] = h[:, :HALF] * dinv[:, None]
    g_ref[1] = h[:, HALF:] * dinv[:, None]


def _mid_body(t_ref, degp_ref, u_ref):
    inv = 1.0 / (jnp.sum(degp_ref[...], axis=0) + 1.0)
    u_ref[0] = t_ref[0] * inv[:, None]
    u_ref[1] = t_ref[1] * inv[:, None]


def _fin_body(v_ref, degp_ref, o_ref):
    dinv = lax.rsqrt(jnp.sum(degp_ref[...], axis=0) + 1.0)
    o_ref[:, :HALF] = v_ref[0] * dinv[:, None]
    o_ref[:, HALF:] = v_ref[1] * dinv[:, None]


_BN = 1000

_lin_call = pl.pallas_call(
    _lin_body,
    grid=(N // _BN,),
    in_specs=[
        pl.BlockSpec((_BN, D_IN), lambda i: (i, 0)),
        pl.BlockSpec((D_IN, D_IN), lambda i: (0, 0)),
        pl.BlockSpec((NC * NS, _BN), lambda i: (0, i)),
    ],
    out_specs=pl.BlockSpec((NC, _BN, HALF), lambda i: (0, i, 0)),
    out_shape=jax.ShapeDtypeStruct((NC, NPAD, HALF), jnp.float32),
)

_mid_call = pl.pallas_call(
    _mid_body,
    grid=(N // _BN,),
    in_specs=[
        pl.BlockSpec((NC, _BN, HALF), lambda i: (0, i, 0)),
        pl.BlockSpec((NC * NS, _BN), lambda i: (0, i)),
    ],
    out_specs=pl.BlockSpec((NC, _BN, HALF), lambda i: (0, i, 0)),
    out_shape=jax.ShapeDtypeStruct((NC, NPAD, HALF), jnp.float32),
)

_fin_call = pl.pallas_call(
    _fin_body,
    grid=(N // _BN,),
    in_specs=[
        pl.BlockSpec((NC, _BN, HALF), lambda i: (0, i, 0)),
        pl.BlockSpec((NC * NS, _BN), lambda i: (0, i)),
    ],
    out_specs=pl.BlockSpec((_BN, D_IN), lambda i: (i, 0)),
    out_shape=jax.ShapeDtypeStruct((N, D_IN), jnp.float32),
)


@jax.jit
def kernel(x, edge_index, W):
    row = edge_index[0].astype(jnp.int32)
    col = edge_index[1].astype(jnp.int32)
    pad = E_PAD - E
    row_p = jnp.concatenate([row, jnp.zeros((pad,), jnp.int32)])
    col_p = jnp.concatenate([col, jnp.full((pad,), DUMMY, jnp.int32)])
    col3 = col_p.reshape(NS, CH_PER_TILE, K_CH)

    degp = _deg_call(col_p)
    g = _lin_call(x, W, degp)
    t = _hop_call(g.reshape(NC * NPAD, HALF), row_p, col3)
    u = _mid_call(t.reshape(NC, NPAD, HALF), degp)
    v = _hop_call(u.reshape(NC * NPAD, HALF), row_p, col3)
    return _fin_call(v.reshape(NC, NPAD, HALF), degp)


# speedup vs baseline: 7.2934x; 7.2934x over previous
import jax
import jax.numpy as jnp
from jax import lax
from jax.experimental import pallas as pl
from jax.experimental.pallas import tpu as pltpu
from jax.experimental.pallas import tpu_sc as plsc

N = 10000
E = 160000
D_IN = 256
HALF = 128
NC, NS = 2, 16
K_CH = 128
CH_PER_TILE = 80
E_TILE = CH_PER_TILE * K_CH
E_PAD = NS * E_TILE
NPAD = 10240
ROWS_PT = NPAD // NS
DUMMY = N

_MESH = plsc.VectorSubcoreMesh(
    core_axis_name="c", subcore_axis_name="s", num_cores=NC, num_subcores=NS
)


DW = 128


def _deg_body(col_hbm, zeros_hbm, ones_hbm, out_hbm, cbuf, obuf, accd):
    c = lax.axis_index("c")
    s = lax.axis_index("s")
    ch_my = CH_PER_TILE // NC
    pltpu.sync_copy(ones_hbm, obuf)
    pltpu.sync_copy(col_hbm.at[s], cbuf)
    pltpu.sync_copy(
        zeros_hbm.at[pl.ds(s * ROWS_PT, ROWS_PT)],
        accd.at[pl.ds(s * ROWS_PT, ROWS_PT)],
    )
    plsc.subcore_barrier()

    def step(j, _):
        pltpu.sync_copy(obuf, accd.at[cbuf.at[j]], add=True)
        return 0

    lax.fori_loop(c * ch_my, (c + 1) * ch_my, step, 0)
    plsc.subcore_barrier()
    pltpu.sync_copy(
        accd.at[pl.ds(s * ROWS_PT, ROWS_PT)],
        out_hbm.at[c, pl.ds(s * ROWS_PT, ROWS_PT)],
    )


_deg_call = pl.kernel(
    _deg_body,
    out_type=jax.ShapeDtypeStruct((NC, NPAD, DW), jnp.float32),
    mesh=_MESH,
    scratch_types=[
        pltpu.VMEM((CH_PER_TILE, K_CH), jnp.int32),
        pltpu.VMEM((K_CH, DW), jnp.float32),
        pltpu.VMEM_SHARED((NPAD, DW), jnp.float32),
    ],
)


def _hop_body(g_hbm, row_hbm, col_hbm, out_hbm, rowbuf, colbuf, gbuf, dsem, acc):
    c = lax.axis_index("c")
    s = lax.axis_index("s")
    pltpu.sync_copy(
        g_hbm.at[pl.ds(c * NPAD + s * ROWS_PT, ROWS_PT)],
        acc.at[pl.ds(s * ROWS_PT, ROWS_PT)],
    )
    pltpu.sync_copy(row_hbm.at[pl.ds(s * E_TILE, E_TILE)], rowbuf)
    pltpu.sync_copy(col_hbm.at[s], colbuf)
    off = c * NPAD

    def adj(i, _):
        rowbuf[pl.ds(i * 16, 16)] = rowbuf[pl.ds(i * 16, 16)] + off
        return 0

    lax.fori_loop(0, E_TILE // 16, adj, 0)
    plsc.subcore_barrier()

    def step(j, _):
        pltpu.async_copy(
            g_hbm.at[rowbuf.at[pl.ds(j * K_CH, K_CH)]], gbuf, dsem
        ).wait()
        pltpu.sync_copy(gbuf, acc.at[colbuf.at[j]], add=True)
        return 0

    lax.fori_loop(0, CH_PER_TILE, step, 0)
    plsc.subcore_barrier()
    pltpu.sync_copy(
        acc.at[pl.ds(s * ROWS_PT, ROWS_PT)],
        out_hbm.at[pl.ds(c * NPAD + s * ROWS_PT, ROWS_PT)],
    )


_hop_call = pl.kernel(
    _hop_body,
    out_type=jax.ShapeDtypeStruct((NC * NPAD, HALF), jnp.float32),
    mesh=_MESH,
    scratch_types=[
        pltpu.VMEM((E_TILE,), jnp.int32),
        pltpu.VMEM((CH_PER_TILE, K_CH), jnp.int32),
        pltpu.VMEM((K_CH, HALF), jnp.float32),
        pltpu.SemaphoreType.DMA,
        pltpu.VMEM_SHARED((NPAD, HALF), jnp.float32),
    ],
)


def _deg_of(degp_ref):
    return degp_ref[0, :, 0:1] + degp_ref[1, :, 0:1] + 1.0


def _lin_body(x_ref, w_ref, degp_ref, g_ref):
    h = lax.dot_general(
        x_ref[...], w_ref[...], (((1,), (1,)), ((), ())),
        preferred_element_type=jnp.float32,
    )
    dinv = lax.rsqrt(_deg_of(degp_ref))
    g_ref[0] = h[:, :HALF] * dinv
    g_ref[1] = h[:, HALF:] * dinv


def _mid_body(t_ref, degp_ref, u_ref):
    inv = 1.0 / _deg_of(degp_ref)
    u_ref[0] = t_ref[0] * inv
    u_ref[1] = t_ref[1] * inv


def _fin_body(v_ref, degp_ref, o_ref):
    dinv = lax.rsqrt(_deg_of(degp_ref))
    o_ref[:, :HALF] = v_ref[0] * dinv
    o_ref[:, HALF:] = v_ref[1] * dinv


_BN = 1024

_lin_call = pl.pallas_call(
    _lin_body,
    grid=(NPAD // _BN,),
    in_specs=[
        pl.BlockSpec((_BN, D_IN), lambda i: (i, 0)),
        pl.BlockSpec((D_IN, D_IN), lambda i: (0, 0)),
        pl.BlockSpec((NC, _BN, DW), lambda i: (0, i, 0)),
    ],
    out_specs=pl.BlockSpec((NC, _BN, HALF), lambda i: (0, i, 0)),
    out_shape=jax.ShapeDtypeStruct((NC, NPAD, HALF), jnp.float32),
)

_mid_call = pl.pallas_call(
    _mid_body,
    grid=(NPAD // _BN,),
    in_specs=[
        pl.BlockSpec((NC, _BN, HALF), lambda i: (0, i, 0)),
        pl.BlockSpec((NC, _BN, DW), lambda i: (0, i, 0)),
    ],
    out_specs=pl.BlockSpec((NC, _BN, HALF), lambda i: (0, i, 0)),
    out_shape=jax.ShapeDtypeStruct((NC, NPAD, HALF), jnp.float32),
)

_fin_call = pl.pallas_call(
    _fin_body,
    grid=(NPAD // _BN,),
    in_specs=[
        pl.BlockSpec((NC, _BN, HALF), lambda i: (0, i, 0)),
        pl.BlockSpec((NC, _BN, DW), lambda i: (0, i, 0)),
    ],
    out_specs=pl.BlockSpec((_BN, D_IN), lambda i: (i, 0)),
    out_shape=jax.ShapeDtypeStruct((N, D_IN), jnp.float32),
)


@jax.jit
def kernel(x, edge_index, W):
    row = edge_index[0].astype(jnp.int32)
    col = edge_index[1].astype(jnp.int32)
    pad = E_PAD - E
    row_p = jnp.concatenate([row, jnp.zeros((pad,), jnp.int32)])
    col_p = jnp.concatenate([col, jnp.full((pad,), DUMMY, jnp.int32)])
    col3 = col_p.reshape(NS, CH_PER_TILE, K_CH)

    degz = jnp.zeros((NPAD, DW), jnp.float32)
    dego = jnp.ones((K_CH, DW), jnp.float32)
    degp = _deg_call(col3, degz, dego)
    g = _lin_call(x, W, degp)
    t = _hop_call(g.reshape(NC * NPAD, HALF), row_p, col3)
    u = _mid_call(t.reshape(NC, NPAD, HALF), degp)
    v = _hop_call(u.reshape(NC * NPAD, HALF), row_p, col3)
    return _fin_call(v.reshape(NC, NPAD, HALF), degp)


# speedup vs baseline: 8.5397x; 1.1709x over previous
import jax
import jax.numpy as jnp
from jax import lax
from jax.experimental import pallas as pl
from jax.experimental.pallas import tpu as pltpu
from jax.experimental.pallas import tpu_sc as plsc

N = 10000
E = 160000
D_IN = 256
HALF = 128
NC, NS = 2, 16
K_CH = 128
CH_PER_TILE = 80
E_TILE = CH_PER_TILE * K_CH
E_PAD = NS * E_TILE
NPAD = 10240
ROWS_PT = NPAD // NS
DUMMY = N

_MESH = plsc.VectorSubcoreMesh(
    core_axis_name="c", subcore_axis_name="s", num_cores=NC, num_subcores=NS
)


DW = 128


def _deg_body(col_hbm, zeros_hbm, ones_hbm, out_hbm, cbuf, obuf, accd):
    c = lax.axis_index("c")
    s = lax.axis_index("s")
    ch_my = CH_PER_TILE // NC
    pltpu.sync_copy(ones_hbm, obuf)
    pltpu.sync_copy(col_hbm.at[s], cbuf)
    pltpu.sync_copy(
        zeros_hbm.at[pl.ds(s * ROWS_PT, ROWS_PT)],
        accd.at[pl.ds(s * ROWS_PT, ROWS_PT)],
    )
    plsc.subcore_barrier()

    def step(j, _):
        pltpu.sync_copy(obuf, accd.at[cbuf.at[j]], add=True)
        return 0

    lax.fori_loop(c * ch_my, (c + 1) * ch_my, step, 0)
    plsc.subcore_barrier()
    pltpu.sync_copy(
        accd.at[pl.ds(s * ROWS_PT, ROWS_PT)],
        out_hbm.at[c, pl.ds(s * ROWS_PT, ROWS_PT)],
    )


_deg_call = pl.kernel(
    _deg_body,
    out_type=jax.ShapeDtypeStruct((NC, NPAD, DW), jnp.float32),
    mesh=_MESH,
    scratch_types=[
        pltpu.VMEM((CH_PER_TILE, K_CH), jnp.int32),
        pltpu.VMEM((K_CH, DW), jnp.float32),
        pltpu.VMEM_SHARED((NPAD, DW), jnp.float32),
    ],
)


def _hop_body(g_hbm, row_hbm, col_hbm, out_hbm, rowbuf, colbuf, gbufs, dsems, acc):
    c = lax.axis_index("c")
    s = lax.axis_index("s")
    pltpu.sync_copy(
        g_hbm.at[pl.ds(c * NPAD + s * ROWS_PT, ROWS_PT)],
        acc.at[pl.ds(s * ROWS_PT, ROWS_PT)],
    )
    pltpu.sync_copy(col_hbm.at[s], colbuf)
    off = c * NPAD
    plsc.subcore_barrier()

    G = 40
    gbuf0, gbuf1 = gbufs
    sem0, sem1 = dsems

    def wait(buf, sem):
        pltpu.make_async_copy(g_hbm.at[pl.ds(0, K_CH)], buf, sem).wait()

    def group(gi, _):
        cb = gi * G
        pltpu.sync_copy(
            row_hbm.at[pl.ds(s * E_TILE + cb * K_CH, G * K_CH)], rowbuf
        )

        def adj(i, _):
            rowbuf[pl.ds(i * 16, 16)] = rowbuf[pl.ds(i * 16, 16)] + off
            return 0

        lax.fori_loop(0, G * K_CH // 16, adj, 0)

        def fire(j, buf, sem):
            pltpu.async_copy(
                g_hbm.at[rowbuf.at[pl.ds(j * K_CH, K_CH)]], buf, sem
            )

        fire(0, gbuf0, sem0)

        def outer(t, _):
            j0 = 2 * t
            fire(j0 + 1, gbuf1, sem1)
            wait(gbuf0, sem0)
            pltpu.sync_copy(gbuf0, acc.at[colbuf.at[cb + j0]], add=True)

            @pl.when(j0 + 2 < G)
            def _():
                fire(j0 + 2, gbuf0, sem0)

            wait(gbuf1, sem1)
            pltpu.sync_copy(gbuf1, acc.at[colbuf.at[cb + j0 + 1]], add=True)
            return 0

        lax.fori_loop(0, G // 2, outer, 0)
        return 0

    lax.fori_loop(0, CH_PER_TILE // G, group, 0)
    plsc.subcore_barrier()
    pltpu.sync_copy(
        acc.at[pl.ds(s * ROWS_PT, ROWS_PT)],
        out_hbm.at[pl.ds(c * NPAD + s * ROWS_PT, ROWS_PT)],
    )


_hop_call = pl.kernel(
    _hop_body,
    out_type=jax.ShapeDtypeStruct((NC * NPAD, HALF), jnp.float32),
    mesh=_MESH,
    scratch_types=[
        pltpu.VMEM((40 * K_CH,), jnp.int32),
        pltpu.VMEM((CH_PER_TILE, K_CH), jnp.int32),
        [pltpu.VMEM((K_CH, HALF), jnp.float32)] * 2,
        [pltpu.SemaphoreType.DMA] * 2,
        pltpu.VMEM_SHARED((NPAD, HALF), jnp.float32),
    ],
)


def _deg_of(degp_ref):
    return degp_ref[0, :, 0:1] + degp_ref[1, :, 0:1] + 1.0


def _lin_body(x_ref, w_ref, degp_ref, g_ref):
    h = lax.dot_general(
        x_ref[...], w_ref[...], (((1,), (1,)), ((), ())),
        preferred_element_type=jnp.float32,
    )
    dinv = lax.rsqrt(_deg_of(degp_ref))
    g_ref[0] = h[:, :HALF] * dinv
    g_ref[1] = h[:, HALF:] * dinv


def _mid_body(t_ref, degp_ref, u_ref):
    inv = 1.0 / _deg_of(degp_ref)
    u_ref[0] = t_ref[0] * inv
    u_ref[1] = t_ref[1] * inv


def _fin_body(v_ref, degp_ref, o_ref):
    dinv = lax.rsqrt(_deg_of(degp_ref))
    o_ref[:, :HALF] = v_ref[0] * dinv
    o_ref[:, HALF:] = v_ref[1] * dinv


_BN = 1024

_lin_call = pl.pallas_call(
    _lin_body,
    grid=(NPAD // _BN,),
    in_specs=[
        pl.BlockSpec((_BN, D_IN), lambda i: (i, 0)),
        pl.BlockSpec((D_IN, D_IN), lambda i: (0, 0)),
        pl.BlockSpec((NC, _BN, DW), lambda i: (0, i, 0)),
    ],
    out_specs=pl.BlockSpec((NC, _BN, HALF), lambda i: (0, i, 0)),
    out_shape=jax.ShapeDtypeStruct((NC, NPAD, HALF), jnp.float32),
)

_mid_call = pl.pallas_call(
    _mid_body,
    grid=(NPAD // _BN,),
    in_specs=[
        pl.BlockSpec((NC, _BN, HALF), lambda i: (0, i, 0)),
        pl.BlockSpec((NC, _BN, DW), lambda i: (0, i, 0)),
    ],
    out_specs=pl.BlockSpec((NC, _BN, HALF), lambda i: (0, i, 0)),
    out_shape=jax.ShapeDtypeStruct((NC, NPAD, HALF), jnp.float32),
)

_fin_call = pl.pallas_call(
    _fin_body,
    grid=(NPAD // _BN,),
    in_specs=[
        pl.BlockSpec((NC, _BN, HALF), lambda i: (0, i, 0)),
        pl.BlockSpec((NC, _BN, DW), lambda i: (0, i, 0)),
    ],
    out_specs=pl.BlockSpec((_BN, D_IN), lambda i: (i, 0)),
    out_shape=jax.ShapeDtypeStruct((N, D_IN), jnp.float32),
)


@jax.jit
def kernel(x, edge_index, W):
    row = edge_index[0].astype(jnp.int32)
    col = edge_index[1].astype(jnp.int32)
    pad = E_PAD - E
    row_p = jnp.concatenate([row, jnp.zeros((pad,), jnp.int32)])
    col_p = jnp.concatenate([col, jnp.full((pad,), DUMMY, jnp.int32)])
    col3 = col_p.reshape(NS, CH_PER_TILE, K_CH)

    degz = jnp.zeros((NPAD, DW), jnp.float32)
    dego = jnp.ones((K_CH, DW), jnp.float32)
    degp = _deg_call(col3, degz, dego)
    g = _lin_call(x, W, degp)
    t = _hop_call(g.reshape(NC * NPAD, HALF), row_p, col3)
    u = _mid_call(t.reshape(NC, NPAD, HALF), degp)
    v = _hop_call(u.reshape(NC * NPAD, HALF), row_p, col3)
    return _fin_call(v.reshape(NC, NPAD, HALF), degp)
